# baseline (speedup 1.0000x reference)
import functools

import jax
import jax.numpy as jnp
from jax import lax
from jax.experimental import pallas as pl
from jax.experimental.pallas import tpu as pltpu
from jax.experimental.pallas import tpu_sc as plsc

PAD = 1
L = 16


def _sc_kernel(inp_flat, weights, bsz, seq, d):
    NC, NS = 2, 16
    NW = NC * NS
    n = bsz * seq
    sl = n // NW
    wpr = NW // bsz
    G = 32
    ng = sl // G
    vpc = G // L
    SL, LN = 8, d // 8

    mesh = plsc.VectorSubcoreMesh(core_axis_name="c", subcore_axis_name="s")

    @functools.partial(
        pl.kernel,
        out_type=jax.ShapeDtypeStruct((n, SL, LN), jnp.float32),
        mesh=mesh,
        scratch_types=[
            pltpu.VMEM((seq,), jnp.int32),
            pltpu.VMEM((sl,), jnp.int32),
            pltpu.VMEM((3 * L,), jnp.int32),
            pltpu.VMEM((3, G, SL, LN), jnp.float32),
            pltpu.SemaphoreType.DMA,
            pltpu.SemaphoreType.DMA,
            pltpu.SemaphoreType.DMA,
            pltpu.SemaphoreType.DMA,
            pltpu.SemaphoreType.DMA,
            pltpu.SemaphoreType.DMA,
        ],
    )
    def k(inp_hbm, tab_hbm, out_hbm, row_v, idx_v, sh_v, rows_v,
          sg0, sg1, sg2, so0, so1, so2):
        wid = lax.axis_index("s") * NC + lax.axis_index("c")
        b = wid // wpr
        c = wid % wpr
        off = c * sl
        base = wid * sl

        pltpu.sync_copy(inp_hbm.at[pl.ds(b * seq, seq)], row_v)

        zero = jnp.zeros((L,), jnp.int32)
        sh_v[pl.ds(0, L)] = zero
        sh_v[pl.ds(2 * L, L)] = zero

        def shift_scans(x):
            p = x
            for kk in (1, 2, 4, 8):
                sh_v[pl.ds(L, L)] = p
                p = p + sh_v[pl.ds(L - kk, L)]
            s = x
            for kk in (1, 2, 4, 8):
                sh_v[pl.ds(L, L)] = s
                s = s + sh_v[pl.ds(L + kk, L)]
            return p, s

        def lane_total_scalar(x):
            t = x[0]
            for u in range(1, L):
                t = t + x[u]
            return t

        def count8(lo, a):
            for u in range(8):
                v = row_v[pl.ds(lo + u * L, L)]
                a = a + jnp.where(v != PAD, 1, 0)
            return a

        acc = lax.fori_loop(0, off // (8 * L),
                            lambda i, a: count8(i * 8 * L, a), zero)
        acc2 = lax.fori_loop(0, sl // (8 * L),
                             lambda i, a: count8(off + i * 8 * L, a), zero)
        prefix = lane_total_scalar(acc)
        stot = lane_total_scalar(acc2)

        p0, s0 = shift_scans(acc)
        carry0 = p0 + s0 - acc

        def chunk_positions(g, carry):
            for t in range(vpc):
                v = row_v[pl.ds(off + g * G + t * L, L)]
                m = jnp.where(v != PAD, 1, 0)
                p, s = shift_scans(m)
                idx_v[pl.ds(g * G + t * L, L)] = (carry + p) * m + PAD
                carry = carry + (p + s - m)
            return carry

        r = [rows_v.at[0], rows_v.at[1], rows_v.at[2]]
        sg = [sg0, sg1, sg2]
        so = [so0, so1, so2]

        def outw(g, j):
            pltpu.async_copy(r[j], out_hbm.at[pl.ds(base + g * G, G)], so[j])

        def wait_g(j):
            pltpu.make_async_copy(tab_hbm.at[pl.ds(0, G)], r[j], sg[j]).wait()

        def wait_o(j):
            pltpu.make_async_copy(r[j], out_hbm.at[pl.ds(base, G)], so[j]).wait()

        def run_ring(gath, with_pos):
            carry = carry0
            if with_pos:
                for j in range(3):
                    carry = chunk_positions(j, carry)
            for j in range(3):
                gath(j, j)

            def body(h, carry):
                g = 3 * h
                if with_pos:
                    for j in range(3):
                        carry = chunk_positions(g + 3 + j, carry)
                for j in range(3):
                    wait_g(j)
                    outw(g + j, j)
                    wait_o(j)
                    gath(g + j + 3, j)
                return carry

            carry = lax.fori_loop(0, (ng - 5) // 3, body, carry)

            gtail = ng - 5
            if with_pos:
                carry = chunk_positions(gtail + 3, carry)
                carry = chunk_positions(gtail + 4, carry)
            for j in range(3):
                wait_g(j)
                outw(gtail + j, j)
                if j < 2:
                    wait_o(j)
                    gath(gtail + j + 3, j)
            for j in range(2):
                wait_g(j)
                outw(ng - 2 + j, j)
            for j in range(3):
                wait_o(j)

        @pl.when(stot == sl)
        def _fast():
            row0 = prefix + 1 + PAD

            def gath_lin(g, j):
                pltpu.async_copy(tab_hbm.at[pl.ds(row0 + g * G, G)],
                                 r[j], sg[j])

            run_ring(gath_lin, with_pos=False)

        @pl.when(stot != sl)
        def _slow():
            def gath_idx(g, j):
                pltpu.async_copy(tab_hbm.at[idx_v.at[pl.ds(g * G, G)]],
                                 r[j], sg[j])

            run_ring(gath_idx, with_pos=True)

    return k(inp_flat, weights.reshape(weights.shape[0], SL, LN))


def kernel(input, weights):
    bsz, seq = input.shape
    nrows, d = weights.shape
    out = _sc_kernel(input.reshape(bsz * seq), weights, bsz, seq, d)
    return lax.stop_gradient(out.reshape(bsz, seq, d))


# speedup vs baseline: 2.5937x; 2.5937x over previous
import functools

import jax
import jax.numpy as jnp
from jax import lax
from jax.experimental import pallas as pl
from jax.experimental.pallas import tpu as pltpu
from jax.experimental.pallas import tpu_sc as plsc

PAD = 1


def _pos_body(inp_ref, out_ref):
    x = inp_ref[...]
    m = jnp.where(x != PAD, 1, 0)
    b, s = x.shape
    cs = m
    k = 1
    while k < s:
        z = jnp.zeros((b, k), jnp.int32)
        cs = cs + jnp.concatenate([z, cs[:, : s - k]], axis=1)
        k *= 2
    out_ref[...] = cs * m + PAD


def _positions(inp):
    return pl.pallas_call(
        _pos_body,
        out_shape=jax.ShapeDtypeStruct(inp.shape, jnp.int32),
    )(inp)


def _sc_gather(positions_flat, weights, n, d):
    NC, NS = 2, 16
    NW = NC * NS
    sl = n // NW
    G = 32
    ng = sl // G

    mesh = plsc.VectorSubcoreMesh(core_axis_name="c", subcore_axis_name="s")

    @functools.partial(
        pl.kernel,
        out_type=jax.ShapeDtypeStruct((n, d), jnp.float32),
        mesh=mesh,
        scratch_types=[
            pltpu.VMEM((sl,), jnp.int32),
            pltpu.VMEM((3, G, d), jnp.float32),
            pltpu.SemaphoreType.DMA,
            pltpu.SemaphoreType.DMA,
            pltpu.SemaphoreType.DMA,
            pltpu.SemaphoreType.DMA,
            pltpu.SemaphoreType.DMA,
            pltpu.SemaphoreType.DMA,
        ],
    )
    def k(idx_hbm, tab_hbm, out_hbm, idx_v, rows_v,
          sg0, sg1, sg2, so0, so1, so2):
        wid = lax.axis_index("s") * NC + lax.axis_index("c")
        base = wid * sl
        pltpu.sync_copy(idx_hbm.at[pl.ds(base, sl)], idx_v)

        r = [rows_v.at[0], rows_v.at[1], rows_v.at[2]]
        sg = [sg0, sg1, sg2]
        so = [so0, so1, so2]

        def gath(g, j):
            pltpu.async_copy(tab_hbm.at[idx_v.at[pl.ds(g * G, G)]], r[j], sg[j])

        def outw(g, j):
            pltpu.async_copy(r[j], out_hbm.at[pl.ds(base + g * G, G)], so[j])

        def wait_g(j):
            pltpu.make_async_copy(tab_hbm.at[pl.ds(0, G)], r[j], sg[j]).wait()

        def wait_o(j):
            pltpu.make_async_copy(r[j], out_hbm.at[pl.ds(base, G)], so[j]).wait()

        for j in range(3):
            gath(j, j)

        def body(h, carry):
            g = 3 * h
            for j in range(3):
                wait_g(j)
                outw(g + j, j)
                wait_o(j)
                gath(g + j + 3, j)
            return carry

        lax.fori_loop(0, (ng - 5) // 3, body, 0)

        gtail = ng - 5
        for j in range(3):
            wait_g(j)
            outw(gtail + j, j)
            if j < 2:
                wait_o(j)
                gath(gtail + j + 3, j)
        for j in range(2):
            wait_g(j)
            outw(ng - 2 + j, j)
        for j in range(3):
            wait_o(j)

    return k(positions_flat, weights)


def kernel(input, weights):
    bsz, seq = input.shape
    nrows, d = weights.shape
    pos = _positions(input).reshape(bsz * seq)
    out = _sc_gather(pos, weights, bsz * seq, d)
    return lax.stop_gradient(out.reshape(bsz, seq, d))
